# Initial kernel scaffold; baseline (speedup 1.0000x reference)
import jax
import jax.numpy as jnp
from jax.experimental import pallas as pl


def kernel(out, y):
    raise NotImplementedError("write your pallas kernel here")


# speedup vs baseline: 4.1033x; 4.1033x over previous
import math

import numpy as np

import jax
import jax.numpy as jnp
from jax.experimental import pallas as pl
from jax.experimental.pallas import tpu as pltpu
from jax.experimental.pallas import tpu_sc as plsc

N = 4096
V = 32000
LS = 0.1
PAD = 0

_EPS = float(np.float32(LS / (V - 1)))
_P = 1.0 - LS
_C1 = _EPS * math.log(_EPS)
_C2 = _P * math.log(_P)

_ROW_BLK = 128
_GATHER_W = 128


def _colsum_body(x_ref, y_ref, cs_ref, g_ref):
    @pl.when(pl.program_id(0) == 0)
    def _init():
        cs_ref[...] = jnp.zeros_like(cs_ref)

    x = x_ref[...]
    cs_ref[...] += jnp.sum(x, axis=0, keepdims=True)
    cols = jax.lax.broadcasted_iota(jnp.int32, (_ROW_BLK, V), 1)
    g_ref[...] = jnp.sum(
        jnp.where(cols == y_ref[...], x, jnp.float32(0.0)),
        axis=1, keepdims=True)


def _combine_body(cs_ref, y_ref, ycol_ref, g_ref, yyrows_ref, o_ref):
    yv = y_ref[...]
    cs = cs_ref[...]
    y_col = ycol_ref[...]
    gv = g_ref[...]
    yyrows = yyrows_ref[...]
    word = jnp.sum((yv != PAD).astype(jnp.float32))
    m = jnp.float32(N) - word
    yc_col = jnp.minimum(y_col, N - 1)
    lanes = jax.lax.broadcasted_iota(jnp.int32, (N, 128), 1)
    yy_col = jnp.sum(jnp.where(lanes == yc_col % 128, yyrows, 0),
                     axis=1, keepdims=True)
    masked = (y_col < N) & (yy_col == PAD)
    K = jnp.float32(N) - jnp.sum(masked.astype(jnp.float32))
    G = jnp.sum(jnp.where(masked, jnp.float32(0.0), gv))
    S_total = jnp.sum(cs)
    S_masked = jnp.sum(jnp.where(yv == PAD, cs[0:32, :], jnp.float32(0.0)))
    kl = (jnp.float32(N) * (jnp.float32(V) - m) * jnp.float32(_C1)
          + K * jnp.float32(_C2 - _C1)
          - jnp.float32(_EPS) * (S_total - S_masked)
          - jnp.float32(_P - _EPS) * G)
    o_ref[...] = (kl / word)[None, None]


def _sc_gather_yy(y_tbl, y_rowidx):
    mesh = plsc.VectorSubcoreMesh(core_axis_name="c", subcore_axis_name="s")

    @pl.kernel(
        out_type=jax.ShapeDtypeStruct((N, 128), jnp.int32),
        mesh=mesh,
    )
    def run(ytbl_hbm, yidx_hbm, yy_hbm):
        def body(yidx_vmem, yy_vmem):
            pltpu.sync_copy(ytbl_hbm.at[yidx_vmem.at[0]], yy_vmem)

        pltpu.emit_pipeline(
            body,
            grid=(N // _GATHER_W,),
            in_specs=[pl.BlockSpec((1, _GATHER_W), lambda i: (0, i))],
            out_specs=[pl.BlockSpec((_GATHER_W, 128), lambda i: (i, 0))],
            core_axis_name=("c", "s"),
            dimension_semantics=(pltpu.PARALLEL,),
        )(yidx_hbm, yy_hbm)

    return run(y_tbl, y_rowidx)


def kernel(out, y):
    y = y.reshape(-1).astype(jnp.int32)
    out2 = out.reshape(N, V)

    y_rowidx = (jnp.minimum(y, N - 1) // 128).reshape(1, N)
    yy_rows = _sc_gather_yy(y.reshape(N // 128, 128), y_rowidx)

    colsum, g = pl.pallas_call(
        _colsum_body,
        grid=(N // _ROW_BLK,),
        in_specs=[
            pl.BlockSpec((_ROW_BLK, V), lambda i: (i, 0)),
            pl.BlockSpec((_ROW_BLK, 1), lambda i: (i, 0)),
        ],
        out_specs=[
            pl.BlockSpec((1, V), lambda i: (0, 0)),
            pl.BlockSpec((_ROW_BLK, 1), lambda i: (i, 0)),
        ],
        out_shape=[
            jax.ShapeDtypeStruct((1, V), jnp.float32),
            jax.ShapeDtypeStruct((N, 1), jnp.float32),
        ],
    )(out2, y.reshape(N, 1))

    res = pl.pallas_call(
        _combine_body,
        in_specs=[
            pl.BlockSpec((V // 128, 128), lambda: (0, 0)),
            pl.BlockSpec((N // 128, 128), lambda: (0, 0)),
            pl.BlockSpec((N, 1), lambda: (0, 0)),
            pl.BlockSpec((N, 1), lambda: (0, 0)),
            pl.BlockSpec((N, 128), lambda: (0, 0)),
        ],
        out_specs=pl.BlockSpec((1, 1), lambda: (0, 0)),
        out_shape=jax.ShapeDtypeStruct((1, 1), jnp.float32),
    )(colsum.reshape(V // 128, 128), y.reshape(N // 128, 128),
      y.reshape(N, 1), g, yy_rows)

    return res[0, 0]


# speedup vs baseline: 7.5106x; 1.8304x over previous
import dataclasses
import math

import numpy as np

import jax
import jax.numpy as jnp
from jax.experimental import pallas as pl
from jax.experimental.pallas import tpu as pltpu
from jax.experimental.pallas import tpu_sc as plsc

N = 4096
V = 32000
LS = 0.1
PAD = 0

_EPS = float(np.float32(LS / (V - 1)))
_P = 1.0 - LS
_C1 = _EPS * math.log(_EPS)
_C2 = _P * math.log(_P)

_ROW_BLK = 128
_GATHER_W = 128


def _colsum_body(x_ref, y_ref, cs_ref, g_ref):
    @pl.when(pl.program_id(0) == 0)
    def _init():
        cs_ref[...] = jnp.zeros_like(cs_ref)

    x = x_ref[...]
    cs_ref[...] += jnp.sum(x, axis=0, keepdims=True)
    cols = jax.lax.broadcasted_iota(jnp.int32, (_ROW_BLK, V), 1)
    g_ref[...] = jnp.sum(
        jnp.where(cols == y_ref[...], x, jnp.float32(0.0)),
        axis=1, keepdims=True)


def _combine_body(cs_ref, y_ref, g_ref, yy_ref, o_ref):
    yv = y_ref[...]
    cs = cs_ref[...]
    gv = g_ref[...]
    yyv = yy_ref[...]
    word = jnp.sum((yv != PAD).astype(jnp.float32))
    m = jnp.float32(N) - word
    masked = (yv < N) & (yyv == PAD)
    K = jnp.float32(N) - jnp.sum(masked.astype(jnp.float32))
    G = jnp.sum(jnp.where(masked, jnp.float32(0.0), gv))
    S_total = jnp.sum(cs)
    S_masked = jnp.sum(jnp.where(yv == PAD, cs[0:32, :], jnp.float32(0.0)))
    kl = (jnp.float32(N) * (jnp.float32(V) - m) * jnp.float32(_C1)
          + K * jnp.float32(_C2 - _C1)
          - jnp.float32(_EPS) * (S_total - S_masked)
          - jnp.float32(_P - _EPS) * G)
    o_ref[...] = (kl / word)[None, None]


def _sc_gather_yy(y_tbl, y_rows):
    mesh = plsc.VectorSubcoreMesh(core_axis_name="c", subcore_axis_name="s")
    cp = pltpu.CompilerParams()
    if "needs_layout_passes" in pltpu.CompilerParams.__dataclass_fields__:
        cp = dataclasses.replace(cp, needs_layout_passes=False)

    @pl.kernel(
        out_type=jax.ShapeDtypeStruct((N // _GATHER_W, _GATHER_W), jnp.int32),
        mesh=mesh,
        compiler_params=cp,
    )
    def run(ytbl_hbm, yrows_hbm, yy_hbm):
        def body(ytbl_vmem, yc_vmem, yy_vmem):
            @pl.loop(0, _GATHER_W, step=16)
            def _(k):
                idx = jnp.minimum(yc_vmem[0, pl.ds(k, 16)], N - 1)
                vals = plsc.load_gather(ytbl_vmem, [jnp.zeros_like(idx), idx])
                yy_vmem[0, pl.ds(k, 16)] = vals

        pltpu.emit_pipeline(
            body,
            grid=(N // _GATHER_W,),
            in_specs=[
                pl.BlockSpec((1, N), lambda i: (0, 0)),
                pl.BlockSpec((1, _GATHER_W), lambda i: (0, i)),
            ],
            out_specs=[pl.BlockSpec((1, _GATHER_W), lambda i: (i, 0))],
            core_axis_name=("c", "s"),
            dimension_semantics=(pltpu.PARALLEL,),
        )(ytbl_hbm, yrows_hbm, yy_hbm)

    return run(y_tbl, y_rows)


def kernel(out, y):
    y = y.reshape(-1).astype(jnp.int32)
    out2 = out.reshape(N, V)

    yy = _sc_gather_yy(y.reshape(1, N), y.reshape(1, N))

    colsum, g = pl.pallas_call(
        _colsum_body,
        grid=(N // _ROW_BLK,),
        in_specs=[
            pl.BlockSpec((_ROW_BLK, V), lambda i: (i, 0)),
            pl.BlockSpec((_ROW_BLK, 1), lambda i: (i, 0)),
        ],
        out_specs=[
            pl.BlockSpec((1, V), lambda i: (0, 0)),
            pl.BlockSpec((_ROW_BLK, 1), lambda i: (i, 0)),
        ],
        out_shape=[
            jax.ShapeDtypeStruct((1, V), jnp.float32),
            jax.ShapeDtypeStruct((N, 1), jnp.float32),
        ],
    )(out2, y.reshape(N, 1))

    res = pl.pallas_call(
        _combine_body,
        in_specs=[
            pl.BlockSpec((V // 128, 128), lambda: (0, 0)),
            pl.BlockSpec((N // 128, 128), lambda: (0, 0)),
            pl.BlockSpec((N // 128, 128), lambda: (0, 0)),
            pl.BlockSpec((N // 128, 128), lambda: (0, 0)),
        ],
        out_specs=pl.BlockSpec((1, 1), lambda: (0, 0)),
        out_shape=jax.ShapeDtypeStruct((1, 1), jnp.float32),
    )(colsum.reshape(V // 128, 128), y.reshape(N // 128, 128),
      g.reshape(N // 128, 128), yy)

    return res[0, 0]
